# Initial kernel scaffold; baseline (speedup 1.0000x reference)
import jax
import jax.numpy as jnp
from jax.experimental import pallas as pl


def kernel(x, edge_index, edge_attr, batch, emb, ln1_g, ln1_b, W1, b1, ln2_g, ln2_b, W2, b2, mW1, mb1, mW2, mb2):
    raise NotImplementedError("write your pallas kernel here")


# speedup vs baseline: 7.0366x; 7.0366x over previous
import functools

import jax
import jax.numpy as jnp
from jax import lax
from jax.experimental import pallas as pl
from jax.experimental.pallas import tpu as pltpu
from jax.experimental.pallas import tpu_sc as plsc

N = 10000
E = 320000
D = 128
G = 64

NC = 2
NS = 16
NT = NC * NS

NPAD = 10240
EPAD = 327680
CH = 128
NCH = EPAD // NT // CH
NPT = NPAD // NT
NPS = NPAD // NS
EPT = NCH * CH
BN = 1024


def _mesh():
    return plsc.VectorSubcoreMesh(core_axis_name="c", subcore_axis_name="s")


def _sc_gather_deg(x3, col3, ew3, emb):
    @functools.partial(
        pl.kernel,
        mesh=_mesh(),
        out_type=(
            jax.ShapeDtypeStruct((NPAD, D), jnp.float32),
            jax.ShapeDtypeStruct((NC, NPAD, 16), jnp.float32),
        ),
        scratch_types=[
            pltpu.VMEM((5, 64), jnp.int32),
            pltpu.VMEM((64, D), jnp.float32),
            pltpu.VMEM((NCH, CH), jnp.int32),
            pltpu.VMEM((EPT,), jnp.float32),
            pltpu.VMEM((CH, 16), jnp.float32),
            pltpu.VMEM_SHARED((NPAD, 16), jnp.float32),
            pltpu.SemaphoreType.DMA,
        ],
        compiler_params=pltpu.CompilerParams(needs_layout_passes=False),
    )
    def k(x_hbm, col_hbm, ew_hbm, emb_hbm, h_out, deg_out,
          xall, grows, colall, ewall, ewrow, deg_sh, sem):
        c = lax.axis_index("c")
        s = lax.axis_index("s")
        tid = c * NS + s

        def zrow(i, _):
            ewrow[i, :] = jnp.zeros((16,), jnp.float32)
            return 0
        lax.fori_loop(0, CH, zrow, 0)
        for i in range(NPS // CH):
            pltpu.sync_copy(ewrow, deg_sh.at[pl.ds(s * NPS + i * CH, CH)])
        plsc.subcore_barrier()

        pltpu.sync_copy(x_hbm.at[tid], xall)
        for t in range(5):
            pltpu.async_copy(emb_hbm.at[xall.at[t]], grows, sem).wait()
            pltpu.sync_copy(grows, h_out.at[pl.ds(tid * NPT + t * 64, 64)])

        pltpu.sync_copy(col_hbm.at[tid], colall)
        pltpu.sync_copy(ew_hbm.at[tid], ewall)

        def chunk(t, _):
            def srow(j, _):
                fi = jnp.zeros((16,), jnp.int32) + (t * CH + j)
                ewrow[j, :] = plsc.load_gather(ewall, [fi])
                return 0
            lax.fori_loop(0, CH, srow, 0)
            pltpu.sync_copy(ewrow, deg_sh.at[colall.at[t]], add=True)
            return 0
        lax.fori_loop(0, NCH, chunk, 0)

        plsc.subcore_barrier()
        pltpu.sync_copy(deg_sh.at[pl.ds(s * NPS, NPS)],
                        deg_out.at[c, pl.ds(s * NPS, NPS)])

    return k(x3, col3, ew3, emb)


def _sc_edge_agg(hlp, row3, col3, ew3):
    @functools.partial(
        pl.kernel,
        mesh=_mesh(),
        out_type=jax.ShapeDtypeStruct((NC, NPAD, D), jnp.float32),
        scratch_types=[
            pltpu.VMEM((NCH, CH), jnp.int32),
            pltpu.VMEM((NCH, CH), jnp.int32),
            pltpu.VMEM((EPT,), jnp.float32),
            pltpu.VMEM((CH, D), jnp.float32),
            pltpu.VMEM_SHARED((NPAD, D), jnp.float32),
            pltpu.SemaphoreType.DMA,
        ],
        compiler_params=pltpu.CompilerParams(needs_layout_passes=False),
    )
    def k(hlp_hbm, row_hbm, col_hbm, ew_hbm, out_hbm,
          rowall, colall, ewall, rows, agg_sh, sem):
        c = lax.axis_index("c")
        s = lax.axis_index("s")
        tid = c * NS + s

        def zrow(i, _):
            for kk in range(D // 16):
                rows[i, pl.ds(16 * kk, 16)] = jnp.zeros((16,), jnp.float32)
            return 0
        lax.fori_loop(0, CH, zrow, 0)
        for i in range(NPS // CH):
            pltpu.sync_copy(rows, agg_sh.at[pl.ds(s * NPS + i * CH, CH)])
        plsc.subcore_barrier()

        pltpu.sync_copy(row_hbm.at[tid], rowall)
        pltpu.sync_copy(col_hbm.at[tid], colall)
        pltpu.sync_copy(ew_hbm.at[tid], ewall)

        def chunk(t, _):
            pltpu.async_copy(hlp_hbm.at[rowall.at[t]], rows, sem).wait()

            def srow(j, _):
                fi = jnp.zeros((16,), jnp.int32) + (t * CH + j)
                sv = plsc.load_gather(ewall, [fi])
                for kk in range(D // 16):
                    rows[j, pl.ds(16 * kk, 16)] = (
                        rows[j, pl.ds(16 * kk, 16)] * sv)
                return 0
            lax.fori_loop(0, CH, srow, 0)

            pltpu.sync_copy(rows, agg_sh.at[colall.at[t]], add=True)
            return 0
        lax.fori_loop(0, NCH, chunk, 0)

        plsc.subcore_barrier()
        pltpu.sync_copy(agg_sh.at[pl.ds(s * NPS, NPS)],
                        out_hbm.at[c, pl.ds(s * NPS, NPS)])

    return k(hlp, row3, col3, ew3)


def _dis_block(degpair):
    deg = 1.0 + degpair[0, :, 0:1] + degpair[1, :, 0:1]
    return lax.rsqrt(deg)


def _ln_block(h, g, b):
    mu = jnp.mean(h, axis=-1, keepdims=True)
    var = jnp.mean((h - mu) ** 2, axis=-1, keepdims=True)
    return (h - mu) / jnp.sqrt(var + 1e-5) * g + b


def _mm(a, b):
    return lax.dot_general(a, b, (((1,), (0,)), ((), ())),
                           precision=lax.Precision.HIGHEST,
                           preferred_element_type=jnp.float32)


def _tc_ln_mm(hgat, degpair, g1, b1, W1):
    def body(h_ref, deg_ref, g_ref, b_ref, w_ref, o_ref):
        dis = _dis_block(deg_ref[...])
        hn = _ln_block(h_ref[...], g_ref[...], b_ref[...])
        o_ref[...] = dis * _mm(hn, w_ref[...])

    return pl.pallas_call(
        body,
        grid=(NPAD // BN,),
        in_specs=[
            pl.BlockSpec((BN, D), lambda i: (i, 0)),
            pl.BlockSpec((NC, BN, 16), lambda i: (0, i, 0)),
            pl.BlockSpec((1, D), lambda i: (0, 0)),
            pl.BlockSpec((1, D), lambda i: (0, 0)),
            pl.BlockSpec((D, D), lambda i: (0, 0)),
        ],
        out_specs=pl.BlockSpec((BN, D), lambda i: (i, 0)),
        out_shape=jax.ShapeDtypeStruct((NPAD, D), jnp.float32),
    )(hgat, degpair, g1, b1, W1)


def _tc_combine_ln_mm(aggpair, hlp, degpair, bias, g2, b2, W2):
    def body(a_ref, hlp_ref, deg_ref, bias_ref, g_ref, b_ref, w_ref, o_ref):
        dis = _dis_block(deg_ref[...])
        a = a_ref[0] + a_ref[1] + hlp_ref[...]
        h2 = jnp.maximum(dis * a + bias_ref[...], 0.0)
        hn = _ln_block(h2, g_ref[...], b_ref[...])
        o_ref[...] = dis * _mm(hn, w_ref[...])

    return pl.pallas_call(
        body,
        grid=(NPAD // BN,),
        in_specs=[
            pl.BlockSpec((NC, BN, D), lambda i: (0, i, 0)),
            pl.BlockSpec((BN, D), lambda i: (i, 0)),
            pl.BlockSpec((NC, BN, 16), lambda i: (0, i, 0)),
            pl.BlockSpec((1, D), lambda i: (0, 0)),
            pl.BlockSpec((1, D), lambda i: (0, 0)),
            pl.BlockSpec((1, D), lambda i: (0, 0)),
            pl.BlockSpec((D, D), lambda i: (0, 0)),
        ],
        out_specs=pl.BlockSpec((BN, D), lambda i: (i, 0)),
        out_shape=jax.ShapeDtypeStruct((NPAD, D), jnp.float32),
    )(aggpair, hlp, degpair, bias, g2, b2, W2)


def _tc_pool_mlp(aggpair, hlp, degpair, bias, batchf, mW1p, mb1p, mW2p, mb2p):
    nb = NPAD // BN

    def body(a_ref, hlp_ref, deg_ref, bias_ref, bt_ref,
             w1_ref, c1_ref, w2_ref, c2_ref, o_ref, pool_ref, cnt_ref):
        i = pl.program_id(0)

        @pl.when(i == 0)
        def _():
            pool_ref[...] = jnp.zeros_like(pool_ref)
            cnt_ref[...] = jnp.zeros_like(cnt_ref)

        dis = _dis_block(deg_ref[...])
        a = a_ref[0] + a_ref[1] + hlp_ref[...]
        h3 = jnp.maximum(dis * a + bias_ref[...], 0.0)

        bt = bt_ref[0]
        gid = lax.broadcasted_iota(jnp.int32, (G, BN), 0)
        onehot = jnp.where(gid == bt, 1.0, 0.0)
        pool_ref[...] += _mm(onehot, h3)
        cnt_ref[...] += jnp.sum(onehot, axis=1, keepdims=True)

        @pl.when(i == nb - 1)
        def _():
            gavg = pool_ref[...] / jnp.maximum(cnt_ref[...], 1.0)
            z = jnp.maximum(_mm(gavg, w1_ref[...]) + c1_ref[...], 0.0)
            o_ref[...] = _mm(z, w2_ref[...]) + c2_ref[...]

    return pl.pallas_call(
        body,
        grid=(nb,),
        in_specs=[
            pl.BlockSpec((NC, BN, D), lambda i: (0, i, 0)),
            pl.BlockSpec((BN, D), lambda i: (i, 0)),
            pl.BlockSpec((NC, BN, 16), lambda i: (0, i, 0)),
            pl.BlockSpec((1, D), lambda i: (0, 0)),
            pl.BlockSpec((1, 1, BN), lambda i: (i, 0, 0)),
            pl.BlockSpec((D, D), lambda i: (0, 0)),
            pl.BlockSpec((1, D), lambda i: (0, 0)),
            pl.BlockSpec((D, D), lambda i: (0, 0)),
            pl.BlockSpec((1, D), lambda i: (0, 0)),
        ],
        out_specs=pl.BlockSpec((G, D), lambda i: (0, 0)),
        out_shape=jax.ShapeDtypeStruct((G, D), jnp.float32),
        scratch_shapes=[
            pltpu.VMEM((G, D), jnp.float32),
            pltpu.VMEM((G, 1), jnp.float32),
        ],
    )(aggpair, hlp, degpair, bias, batchf, mW1p, mb1p, mW2p, mb2p)


def kernel(x, edge_index, edge_attr, batch, emb, ln1_g, ln1_b, W1, b1,
           ln2_g, ln2_b, W2, b2, mW1, mb1, mW2, mb2):
    i32 = jnp.int32
    f32 = jnp.float32

    x3 = jnp.concatenate(
        [x.astype(i32), jnp.zeros((NPAD - N,), i32)]).reshape(NT, 5, 64)
    row = edge_index[0].astype(i32)
    col = edge_index[1].astype(i32)
    epad = EPAD - E
    row3 = jnp.concatenate([row, jnp.zeros((epad,), i32)]).reshape(NT, NCH, CH)
    col3 = jnp.concatenate([col, jnp.zeros((epad,), i32)]).reshape(NT, NCH, CH)
    ew3 = jnp.concatenate(
        [edge_attr.astype(f32), jnp.zeros((epad,), f32)]).reshape(NT, EPT)
    batchf = jnp.concatenate(
        [batch.astype(i32), jnp.full((NPAD - N,), 65536, i32)]
    ).reshape(NPAD // BN, 1, BN)

    g1 = ln1_g.reshape(1, D)
    c1 = ln1_b.reshape(1, D)
    g2 = ln2_g.reshape(1, D)
    c2 = ln2_b.reshape(1, D)
    b1r = b1.reshape(1, D)
    b2r = b2.reshape(1, D)
    mW1p = jnp.pad(mW1, ((0, 0), (0, D - mW1.shape[1])))
    mb1p = jnp.pad(mb1, (0, D - mb1.shape[0])).reshape(1, D)
    mW2p = jnp.pad(mW2, ((0, D - mW2.shape[0]), (0, D - mW2.shape[1])))
    mb2p = jnp.pad(mb2, (0, D - mb2.shape[0])).reshape(1, D)

    hgat, degpair = _sc_gather_deg(x3, col3, ew3, emb)
    hl1p = _tc_ln_mm(hgat, degpair, g1, c1, W1)
    agg1 = _sc_edge_agg(hl1p, row3, col3, ew3)
    hl2p = _tc_combine_ln_mm(agg1, hl1p, degpair, b1r, g2, c2, W2)
    agg2 = _sc_edge_agg(hl2p, row3, col3, ew3)
    outp = _tc_pool_mlp(agg2, hl2p, degpair, b2r, batchf,
                        mW1p, mb1p, mW2p, mb2p)
    return outp[:, :mW2.shape[1]]


# speedup vs baseline: 7.2629x; 1.0322x over previous
import functools

import jax
import jax.numpy as jnp
from jax import lax
from jax.experimental import pallas as pl
from jax.experimental.pallas import tpu as pltpu
from jax.experimental.pallas import tpu_sc as plsc

N = 10000
E = 320000
D = 128
G = 64

NC = 2
NS = 16
NT = NC * NS

NPAD = 10240
EPAD = 327680
CH = 128
NCH = EPAD // NT // CH
NPT = NPAD // NT
NPS = NPAD // NS
EPT = NCH * CH
BN = 1024


def _mesh():
    return plsc.VectorSubcoreMesh(core_axis_name="c", subcore_axis_name="s")


def _sc_gather_deg(x3, col3, ew3, emb):
    @functools.partial(
        pl.kernel,
        mesh=_mesh(),
        out_type=(
            jax.ShapeDtypeStruct((NPAD, D), jnp.float32),
            jax.ShapeDtypeStruct((NC, NPAD, 16), jnp.float32),
        ),
        scratch_types=[
            pltpu.VMEM((5, 64), jnp.int32),
            pltpu.VMEM((64, D), jnp.float32),
            pltpu.VMEM((SUP, CH), jnp.int32),
            pltpu.VMEM((SUP * CH,), jnp.float32),
            pltpu.VMEM((CH, 16), jnp.float32),
            pltpu.VMEM_SHARED((NPAD, 16), jnp.float32),
            pltpu.SemaphoreType.DMA,
        ],
        compiler_params=pltpu.CompilerParams(needs_layout_passes=False),
    )
    def k(x_hbm, col_hbm, ew_hbm, emb_hbm, h_out, deg_out,
          xall, grows, colall, ewall, ewrow, deg_sh, sem):
        c = lax.axis_index("c")
        s = lax.axis_index("s")
        tid = c * NS + s

        def zrow(i, _):
            ewrow[i, :] = jnp.zeros((16,), jnp.float32)
            return 0
        lax.fori_loop(0, CH, zrow, 0)
        for i in range(NPS // CH):
            pltpu.sync_copy(ewrow, deg_sh.at[pl.ds(s * NPS + i * CH, CH)])
        plsc.subcore_barrier()

        pltpu.sync_copy(x_hbm.at[tid], xall)
        for t in range(5):
            pltpu.async_copy(emb_hbm.at[xall.at[t]], grows, sem).wait()
            pltpu.sync_copy(grows, h_out.at[pl.ds(tid * NPT + t * 64, 64)])

        def super_chunk(ss, _):
            pltpu.sync_copy(col_hbm.at[tid, ss], colall)
            pltpu.sync_copy(ew_hbm.at[tid, ss], ewall)

            def chunk(t, _):
                def srow(j, _):
                    fi = jnp.zeros((16,), jnp.int32) + (t * CH + j)
                    ewrow[j, :] = plsc.load_gather(ewall, [fi])
                    return 0
                lax.fori_loop(0, CH, srow, 0)
                pltpu.sync_copy(ewrow, deg_sh.at[colall.at[t]], add=True)
                return 0
            lax.fori_loop(0, SUP, chunk, 0)
            return 0
        lax.fori_loop(0, NSUP, super_chunk, 0)

        plsc.subcore_barrier()
        pltpu.sync_copy(deg_sh.at[pl.ds(s * NPS, NPS)],
                        deg_out.at[c, pl.ds(s * NPS, NPS)])

    return k(x3, col3, ew3, emb)


SUP = 20
NSUP = NCH // SUP


def _sc_edge_agg(hlp, row3, col3, ew3):
    @functools.partial(
        pl.kernel,
        mesh=_mesh(),
        out_type=jax.ShapeDtypeStruct((NC, NPAD, D), jnp.float32),
        scratch_types=[
            pltpu.VMEM((SUP, CH), jnp.int32),
            pltpu.VMEM((SUP, CH), jnp.int32),
            pltpu.VMEM((SUP * CH,), jnp.float32),
            pltpu.VMEM((CH, D), jnp.float32),
            pltpu.VMEM((CH, D), jnp.float32),
            pltpu.VMEM_SHARED((NPAD, D), jnp.float32),
            pltpu.SemaphoreType.DMA,
            pltpu.SemaphoreType.DMA,
        ],
        compiler_params=pltpu.CompilerParams(needs_layout_passes=False),
    )
    def k(hlp_hbm, row_hbm, col_hbm, ew_hbm, out_hbm,
          rowsup, colsup, ewsup, rows_a, rows_b, agg_sh, sem_a, sem_b):
        c = lax.axis_index("c")
        s = lax.axis_index("s")
        tid = c * NS + s

        def zrow(i, _):
            for kk in range(D // 16):
                rows_a[i, pl.ds(16 * kk, 16)] = jnp.zeros((16,), jnp.float32)
            return 0
        lax.fori_loop(0, CH, zrow, 0)
        for i in range(NPS // CH):
            pltpu.sync_copy(rows_a, agg_sh.at[pl.ds(s * NPS + i * CH, CH)])
        plsc.subcore_barrier()

        def scale(rows, t):
            def srow(j, _):
                fi = jnp.zeros((16,), jnp.int32) + (t * CH + j)
                sv = plsc.load_gather(ewsup, [fi])
                for kk in range(D // 16):
                    rows[j, pl.ds(16 * kk, 16)] = (
                        rows[j, pl.ds(16 * kk, 16)] * sv)
                return 0
            lax.fori_loop(0, CH, srow, 0)

        def super_chunk(ss, _):
            pltpu.sync_copy(row_hbm.at[tid, ss], rowsup)
            pltpu.sync_copy(col_hbm.at[tid, ss], colsup)
            pltpu.sync_copy(ew_hbm.at[tid, ss], ewsup)

            pltpu.async_copy(hlp_hbm.at[rowsup.at[0]], rows_a, sem_a).wait()

            def pair(tt, _):
                t0 = 2 * tt
                t1 = t0 + 1
                t2 = jnp.minimum(t0 + 2, SUP - 1)
                d1 = pltpu.async_copy(
                    hlp_hbm.at[rowsup.at[t1]], rows_b, sem_b)
                scale(rows_a, t0)
                d1.wait()
                pltpu.sync_copy(rows_a, agg_sh.at[colsup.at[t0]], add=True)
                d2 = pltpu.async_copy(
                    hlp_hbm.at[rowsup.at[t2]], rows_a, sem_a)
                scale(rows_b, t1)
                d2.wait()
                pltpu.sync_copy(rows_b, agg_sh.at[colsup.at[t1]], add=True)
                return 0
            lax.fori_loop(0, SUP // 2, pair, 0)
            return 0
        lax.fori_loop(0, NSUP, super_chunk, 0)

        plsc.subcore_barrier()
        pltpu.sync_copy(agg_sh.at[pl.ds(s * NPS, NPS)],
                        out_hbm.at[c, pl.ds(s * NPS, NPS)])

    return k(hlp, row3, col3, ew3)


def _dis_block(degpair):
    deg = 1.0 + degpair[0, :, 0:1] + degpair[1, :, 0:1]
    return lax.rsqrt(deg)


def _ln_block(h, g, b):
    mu = jnp.mean(h, axis=-1, keepdims=True)
    var = jnp.mean((h - mu) ** 2, axis=-1, keepdims=True)
    return (h - mu) / jnp.sqrt(var + 1e-5) * g + b


def _mm(a, b):
    return lax.dot_general(a, b, (((1,), (0,)), ((), ())),
                           precision=lax.Precision.HIGHEST,
                           preferred_element_type=jnp.float32)


def _tc_ln_mm(hgat, degpair, g1, b1, W1):
    def body(h_ref, deg_ref, g_ref, b_ref, w_ref, o_ref):
        dis = _dis_block(deg_ref[...])
        hn = _ln_block(h_ref[...], g_ref[...], b_ref[...])
        o_ref[...] = dis * _mm(hn, w_ref[...])

    return pl.pallas_call(
        body,
        grid=(NPAD // BN,),
        in_specs=[
            pl.BlockSpec((BN, D), lambda i: (i, 0)),
            pl.BlockSpec((NC, BN, 16), lambda i: (0, i, 0)),
            pl.BlockSpec((1, D), lambda i: (0, 0)),
            pl.BlockSpec((1, D), lambda i: (0, 0)),
            pl.BlockSpec((D, D), lambda i: (0, 0)),
        ],
        out_specs=pl.BlockSpec((BN, D), lambda i: (i, 0)),
        out_shape=jax.ShapeDtypeStruct((NPAD, D), jnp.float32),
    )(hgat, degpair, g1, b1, W1)


def _tc_combine_ln_mm(aggpair, hlp, degpair, bias, g2, b2, W2):
    def body(a_ref, hlp_ref, deg_ref, bias_ref, g_ref, b_ref, w_ref, o_ref):
        dis = _dis_block(deg_ref[...])
        a = a_ref[0] + a_ref[1] + hlp_ref[...]
        h2 = jnp.maximum(dis * a + bias_ref[...], 0.0)
        hn = _ln_block(h2, g_ref[...], b_ref[...])
        o_ref[...] = dis * _mm(hn, w_ref[...])

    return pl.pallas_call(
        body,
        grid=(NPAD // BN,),
        in_specs=[
            pl.BlockSpec((NC, BN, D), lambda i: (0, i, 0)),
            pl.BlockSpec((BN, D), lambda i: (i, 0)),
            pl.BlockSpec((NC, BN, 16), lambda i: (0, i, 0)),
            pl.BlockSpec((1, D), lambda i: (0, 0)),
            pl.BlockSpec((1, D), lambda i: (0, 0)),
            pl.BlockSpec((1, D), lambda i: (0, 0)),
            pl.BlockSpec((D, D), lambda i: (0, 0)),
        ],
        out_specs=pl.BlockSpec((BN, D), lambda i: (i, 0)),
        out_shape=jax.ShapeDtypeStruct((NPAD, D), jnp.float32),
    )(aggpair, hlp, degpair, bias, g2, b2, W2)


def _tc_pool_mlp(aggpair, hlp, degpair, bias, batchf, mW1p, mb1p, mW2p, mb2p):
    nb = NPAD // BN

    def body(a_ref, hlp_ref, deg_ref, bias_ref, bt_ref,
             w1_ref, c1_ref, w2_ref, c2_ref, o_ref, pool_ref, cnt_ref):
        i = pl.program_id(0)

        @pl.when(i == 0)
        def _():
            pool_ref[...] = jnp.zeros_like(pool_ref)
            cnt_ref[...] = jnp.zeros_like(cnt_ref)

        dis = _dis_block(deg_ref[...])
        a = a_ref[0] + a_ref[1] + hlp_ref[...]
        h3 = jnp.maximum(dis * a + bias_ref[...], 0.0)

        bt = bt_ref[0]
        gid = lax.broadcasted_iota(jnp.int32, (G, BN), 0)
        onehot = jnp.where(gid == bt, 1.0, 0.0)
        pool_ref[...] += _mm(onehot, h3)
        cnt_ref[...] += jnp.sum(onehot, axis=1, keepdims=True)

        @pl.when(i == nb - 1)
        def _():
            gavg = pool_ref[...] / jnp.maximum(cnt_ref[...], 1.0)
            z = jnp.maximum(_mm(gavg, w1_ref[...]) + c1_ref[...], 0.0)
            o_ref[...] = _mm(z, w2_ref[...]) + c2_ref[...]

    return pl.pallas_call(
        body,
        grid=(nb,),
        in_specs=[
            pl.BlockSpec((NC, BN, D), lambda i: (0, i, 0)),
            pl.BlockSpec((BN, D), lambda i: (i, 0)),
            pl.BlockSpec((NC, BN, 16), lambda i: (0, i, 0)),
            pl.BlockSpec((1, D), lambda i: (0, 0)),
            pl.BlockSpec((1, 1, BN), lambda i: (i, 0, 0)),
            pl.BlockSpec((D, D), lambda i: (0, 0)),
            pl.BlockSpec((1, D), lambda i: (0, 0)),
            pl.BlockSpec((D, D), lambda i: (0, 0)),
            pl.BlockSpec((1, D), lambda i: (0, 0)),
        ],
        out_specs=pl.BlockSpec((G, D), lambda i: (0, 0)),
        out_shape=jax.ShapeDtypeStruct((G, D), jnp.float32),
        scratch_shapes=[
            pltpu.VMEM((G, D), jnp.float32),
            pltpu.VMEM((G, 1), jnp.float32),
        ],
    )(aggpair, hlp, degpair, bias, batchf, mW1p, mb1p, mW2p, mb2p)


def kernel(x, edge_index, edge_attr, batch, emb, ln1_g, ln1_b, W1, b1,
           ln2_g, ln2_b, W2, b2, mW1, mb1, mW2, mb2):
    i32 = jnp.int32
    f32 = jnp.float32

    x3 = jnp.concatenate(
        [x.astype(i32), jnp.zeros((NPAD - N,), i32)]).reshape(NT, 5, 64)
    row = edge_index[0].astype(i32)
    col = edge_index[1].astype(i32)
    epad = EPAD - E
    row3 = jnp.concatenate(
        [row, jnp.zeros((epad,), i32)]).reshape(NT, NSUP, SUP, CH)
    col3 = jnp.concatenate(
        [col, jnp.zeros((epad,), i32)]).reshape(NT, NSUP, SUP, CH)
    ew3 = jnp.concatenate(
        [edge_attr.astype(f32), jnp.zeros((epad,), f32)]
    ).reshape(NT, NSUP, SUP * CH)
    batchf = jnp.concatenate(
        [batch.astype(i32), jnp.full((NPAD - N,), 65536, i32)]
    ).reshape(NPAD // BN, 1, BN)

    g1 = ln1_g.reshape(1, D)
    c1 = ln1_b.reshape(1, D)
    g2 = ln2_g.reshape(1, D)
    c2 = ln2_b.reshape(1, D)
    b1r = b1.reshape(1, D)
    b2r = b2.reshape(1, D)
    mW1p = jnp.pad(mW1, ((0, 0), (0, D - mW1.shape[1])))
    mb1p = jnp.pad(mb1, (0, D - mb1.shape[0])).reshape(1, D)
    mW2p = jnp.pad(mW2, ((0, D - mW2.shape[0]), (0, D - mW2.shape[1])))
    mb2p = jnp.pad(mb2, (0, D - mb2.shape[0])).reshape(1, D)

    hgat, degpair = _sc_gather_deg(x3, col3, ew3, emb)
    hl1p = _tc_ln_mm(hgat, degpair, g1, c1, W1)
    agg1 = _sc_edge_agg(hl1p, row3, col3, ew3)
    hl2p = _tc_combine_ln_mm(agg1, hl1p, degpair, b1r, g2, c2, W2)
    agg2 = _sc_edge_agg(hl2p, row3, col3, ew3)
    outp = _tc_pool_mlp(agg2, hl2p, degpair, b2r, batchf,
                        mW1p, mb1p, mW2p, mb2p)
    return outp[:, :mW2.shape[1]]


# speedup vs baseline: 17.2115x; 2.3698x over previous
import functools

import jax
import jax.numpy as jnp
from jax import lax
from jax.experimental import pallas as pl
from jax.experimental.pallas import tpu as pltpu
from jax.experimental.pallas import tpu_sc as plsc

N = 10000
E = 320000
D = 128
G = 64

NC = 2
NS = 16
NT = NC * NS

NPAD = 10240
EPAD = 327680
CH = 128
NCH = EPAD // NT // CH
NPT = NPAD // NT
NPS = NPAD // NS
EPT = NCH * CH
BN = 1024


def _mesh():
    return plsc.VectorSubcoreMesh(core_axis_name="c", subcore_axis_name="s")


def _sc_gather_deg(x3, col3, ew3, emb):
    @functools.partial(
        pl.kernel,
        mesh=_mesh(),
        out_type=(
            jax.ShapeDtypeStruct((NPAD, D), jnp.float32),
            jax.ShapeDtypeStruct((NC, NPAD, 16), jnp.float32),
        ),
        scratch_types=[
            pltpu.VMEM((5, 64), jnp.int32),
            pltpu.VMEM((64, D), jnp.float32),
            pltpu.VMEM((SUP, CH), jnp.int32),
            pltpu.VMEM((SUP * CH,), jnp.float32),
            pltpu.VMEM((CH, 16), jnp.float32),
            pltpu.VMEM_SHARED((NPAD, 16), jnp.float32),
            pltpu.SemaphoreType.DMA,
        ],
        compiler_params=pltpu.CompilerParams(needs_layout_passes=False),
    )
    def k(x_hbm, col_hbm, ew_hbm, emb_hbm, h_out, deg_out,
          xall, grows, colall, ewall, ewrow, deg_sh, sem):
        c = lax.axis_index("c")
        s = lax.axis_index("s")
        tid = c * NS + s

        def zrow(i, _):
            ewrow[i, :] = jnp.zeros((16,), jnp.float32)
            return 0
        lax.fori_loop(0, CH, zrow, 0)
        for i in range(NPS // CH):
            pltpu.sync_copy(ewrow, deg_sh.at[pl.ds(s * NPS + i * CH, CH)])
        plsc.subcore_barrier()

        pltpu.sync_copy(x_hbm.at[tid], xall)
        for t in range(5):
            pltpu.async_copy(emb_hbm.at[xall.at[t]], grows, sem).wait()
            pltpu.sync_copy(grows, h_out.at[pl.ds(tid * NPT + t * 64, 64)])

        def super_chunk(ss, _):
            pltpu.sync_copy(col_hbm.at[tid, ss], colall)
            pltpu.sync_copy(ew_hbm.at[tid, ss], ewall)

            def chunk(t, _):
                def srow(j, _):
                    fi = jnp.zeros((16,), jnp.int32) + (t * CH + j)
                    ewrow[j, :] = plsc.load_gather(ewall, [fi])
                    return 0
                lax.fori_loop(0, CH, srow, 0)
                pltpu.sync_copy(ewrow, deg_sh.at[colall.at[t]], add=True)
                return 0
            lax.fori_loop(0, SUP, chunk, 0)
            return 0
        lax.fori_loop(0, NSUP, super_chunk, 0)

        plsc.subcore_barrier()
        pltpu.sync_copy(deg_sh.at[pl.ds(s * NPS, NPS)],
                        deg_out.at[c, pl.ds(s * NPS, NPS)])

    return k(x3, col3, ew3, emb)


SUP = 20
NSUP = NCH // SUP


def _sc_edge_agg(hlp, row3, col3, ew3):
    @functools.partial(
        pl.kernel,
        mesh=_mesh(),
        out_type=jax.ShapeDtypeStruct((NC, NPAD, D), jnp.float32),
        scratch_types=[
            pltpu.VMEM((SUP, CH), jnp.int32),
            pltpu.VMEM((SUP, CH), jnp.int32),
            pltpu.VMEM((SUP * CH,), jnp.float32),
            pltpu.VMEM((CH, D), jnp.float32),
            pltpu.VMEM((CH, D), jnp.float32),
            pltpu.VMEM_SHARED((NPAD, D), jnp.float32),
            pltpu.SemaphoreType.DMA,
            pltpu.SemaphoreType.DMA,
        ],
        compiler_params=pltpu.CompilerParams(needs_layout_passes=False),
    )
    def k(hlp_hbm, row_hbm, col_hbm, ew_hbm, out_hbm,
          rowsup, colsup, ewsup, rows_a, rows_b, agg_sh, sem_a, sem_b):
        c = lax.axis_index("c")
        s = lax.axis_index("s")
        tid = c * NS + s

        def zrow(i, _):
            for kk in range(D // 16):
                rows_a[i, pl.ds(16 * kk, 16)] = jnp.zeros((16,), jnp.float32)
            return 0
        lax.fori_loop(0, CH, zrow, 0)
        for i in range(NPS // CH):
            pltpu.sync_copy(rows_a, agg_sh.at[pl.ds(s * NPS + i * CH, CH)])
        plsc.subcore_barrier()

        def scale(rows, t):
            @plsc.parallel_loop(0, CH, unroll=4)
            def _(j):
                fi = jnp.zeros((16,), jnp.int32) + (t * CH + j)
                sv = plsc.load_gather(ewsup, [fi])
                for kk in range(D // 16):
                    rows[j, pl.ds(16 * kk, 16)] = (
                        rows[j, pl.ds(16 * kk, 16)] * sv)

        def super_chunk(ss, _):
            pltpu.sync_copy(row_hbm.at[tid, ss], rowsup)
            pltpu.sync_copy(col_hbm.at[tid, ss], colsup)
            pltpu.sync_copy(ew_hbm.at[tid, ss], ewsup)

            pltpu.async_copy(hlp_hbm.at[rowsup.at[0]], rows_a, sem_a).wait()

            def pair(tt, _):
                t0 = 2 * tt
                t1 = t0 + 1
                t2 = jnp.minimum(t0 + 2, SUP - 1)
                d1 = pltpu.async_copy(
                    hlp_hbm.at[rowsup.at[t1]], rows_b, sem_b)
                scale(rows_a, t0)
                d1.wait()
                pltpu.sync_copy(rows_a, agg_sh.at[colsup.at[t0]], add=True)
                d2 = pltpu.async_copy(
                    hlp_hbm.at[rowsup.at[t2]], rows_a, sem_a)
                scale(rows_b, t1)
                d2.wait()
                pltpu.sync_copy(rows_b, agg_sh.at[colsup.at[t1]], add=True)
                return 0
            lax.fori_loop(0, SUP // 2, pair, 0)
            return 0
        lax.fori_loop(0, NSUP, super_chunk, 0)

        plsc.subcore_barrier()
        pltpu.sync_copy(agg_sh.at[pl.ds(s * NPS, NPS)],
                        out_hbm.at[c, pl.ds(s * NPS, NPS)])

    return k(hlp, row3, col3, ew3)


def _dis_block(degpair):
    deg = 1.0 + degpair[0, :, 0:1] + degpair[1, :, 0:1]
    return lax.rsqrt(deg)


def _ln_block(h, g, b):
    mu = jnp.mean(h, axis=-1, keepdims=True)
    var = jnp.mean((h - mu) ** 2, axis=-1, keepdims=True)
    return (h - mu) / jnp.sqrt(var + 1e-5) * g + b


def _mm(a, b):
    return lax.dot_general(a, b, (((1,), (0,)), ((), ())),
                           precision=lax.Precision.HIGHEST,
                           preferred_element_type=jnp.float32)


def _tc_ln_mm(hgat, degpair, g1, b1, W1):
    def body(h_ref, deg_ref, g_ref, b_ref, w_ref, o_ref):
        dis = _dis_block(deg_ref[...])
        hn = _ln_block(h_ref[...], g_ref[...], b_ref[...])
        o_ref[...] = dis * _mm(hn, w_ref[...])

    return pl.pallas_call(
        body,
        grid=(NPAD // BN,),
        in_specs=[
            pl.BlockSpec((BN, D), lambda i: (i, 0)),
            pl.BlockSpec((NC, BN, 16), lambda i: (0, i, 0)),
            pl.BlockSpec((1, D), lambda i: (0, 0)),
            pl.BlockSpec((1, D), lambda i: (0, 0)),
            pl.BlockSpec((D, D), lambda i: (0, 0)),
        ],
        out_specs=pl.BlockSpec((BN, D), lambda i: (i, 0)),
        out_shape=jax.ShapeDtypeStruct((NPAD, D), jnp.float32),
    )(hgat, degpair, g1, b1, W1)


def _tc_combine_ln_mm(aggpair, hlp, degpair, bias, g2, b2, W2):
    def body(a_ref, hlp_ref, deg_ref, bias_ref, g_ref, b_ref, w_ref, o_ref):
        dis = _dis_block(deg_ref[...])
        a = a_ref[0] + a_ref[1] + hlp_ref[...]
        h2 = jnp.maximum(dis * a + bias_ref[...], 0.0)
        hn = _ln_block(h2, g_ref[...], b_ref[...])
        o_ref[...] = dis * _mm(hn, w_ref[...])

    return pl.pallas_call(
        body,
        grid=(NPAD // BN,),
        in_specs=[
            pl.BlockSpec((NC, BN, D), lambda i: (0, i, 0)),
            pl.BlockSpec((BN, D), lambda i: (i, 0)),
            pl.BlockSpec((NC, BN, 16), lambda i: (0, i, 0)),
            pl.BlockSpec((1, D), lambda i: (0, 0)),
            pl.BlockSpec((1, D), lambda i: (0, 0)),
            pl.BlockSpec((1, D), lambda i: (0, 0)),
            pl.BlockSpec((D, D), lambda i: (0, 0)),
        ],
        out_specs=pl.BlockSpec((BN, D), lambda i: (i, 0)),
        out_shape=jax.ShapeDtypeStruct((NPAD, D), jnp.float32),
    )(aggpair, hlp, degpair, bias, g2, b2, W2)


def _tc_pool_mlp(aggpair, hlp, degpair, bias, batchf, mW1p, mb1p, mW2p, mb2p):
    nb = NPAD // BN

    def body(a_ref, hlp_ref, deg_ref, bias_ref, bt_ref,
             w1_ref, c1_ref, w2_ref, c2_ref, o_ref, pool_ref, cnt_ref):
        i = pl.program_id(0)

        @pl.when(i == 0)
        def _():
            pool_ref[...] = jnp.zeros_like(pool_ref)
            cnt_ref[...] = jnp.zeros_like(cnt_ref)

        dis = _dis_block(deg_ref[...])
        a = a_ref[0] + a_ref[1] + hlp_ref[...]
        h3 = jnp.maximum(dis * a + bias_ref[...], 0.0)

        bt = bt_ref[0]
        gid = lax.broadcasted_iota(jnp.int32, (G, BN), 0)
        onehot = jnp.where(gid == bt, 1.0, 0.0)
        pool_ref[...] += _mm(onehot, h3)
        cnt_ref[...] += jnp.sum(onehot, axis=1, keepdims=True)

        @pl.when(i == nb - 1)
        def _():
            gavg = pool_ref[...] / jnp.maximum(cnt_ref[...], 1.0)
            z = jnp.maximum(_mm(gavg, w1_ref[...]) + c1_ref[...], 0.0)
            o_ref[...] = _mm(z, w2_ref[...]) + c2_ref[...]

    return pl.pallas_call(
        body,
        grid=(nb,),
        in_specs=[
            pl.BlockSpec((NC, BN, D), lambda i: (0, i, 0)),
            pl.BlockSpec((BN, D), lambda i: (i, 0)),
            pl.BlockSpec((NC, BN, 16), lambda i: (0, i, 0)),
            pl.BlockSpec((1, D), lambda i: (0, 0)),
            pl.BlockSpec((1, 1, BN), lambda i: (i, 0, 0)),
            pl.BlockSpec((D, D), lambda i: (0, 0)),
            pl.BlockSpec((1, D), lambda i: (0, 0)),
            pl.BlockSpec((D, D), lambda i: (0, 0)),
            pl.BlockSpec((1, D), lambda i: (0, 0)),
        ],
        out_specs=pl.BlockSpec((G, D), lambda i: (0, 0)),
        out_shape=jax.ShapeDtypeStruct((G, D), jnp.float32),
        scratch_shapes=[
            pltpu.VMEM((G, D), jnp.float32),
            pltpu.VMEM((G, 1), jnp.float32),
        ],
    )(aggpair, hlp, degpair, bias, batchf, mW1p, mb1p, mW2p, mb2p)


def kernel(x, edge_index, edge_attr, batch, emb, ln1_g, ln1_b, W1, b1,
           ln2_g, ln2_b, W2, b2, mW1, mb1, mW2, mb2):
    i32 = jnp.int32
    f32 = jnp.float32

    x3 = jnp.concatenate(
        [x.astype(i32), jnp.zeros((NPAD - N,), i32)]).reshape(NT, 5, 64)
    row = edge_index[0].astype(i32)
    col = edge_index[1].astype(i32)
    epad = EPAD - E
    spread = jnp.arange(epad, dtype=i32) % N
    row3 = jnp.concatenate([row, spread]).reshape(NT, NSUP, SUP, CH)
    col3 = jnp.concatenate([col, spread]).reshape(NT, NSUP, SUP, CH)
    ew3 = jnp.concatenate(
        [edge_attr.astype(f32), jnp.zeros((epad,), f32)]
    ).reshape(NT, NSUP, SUP * CH)
    batchf = jnp.concatenate(
        [batch.astype(i32), jnp.full((NPAD - N,), 65536, i32)]
    ).reshape(NPAD // BN, 1, BN)

    g1 = ln1_g.reshape(1, D)
    c1 = ln1_b.reshape(1, D)
    g2 = ln2_g.reshape(1, D)
    c2 = ln2_b.reshape(1, D)
    b1r = b1.reshape(1, D)
    b2r = b2.reshape(1, D)
    mW1p = jnp.pad(mW1, ((0, 0), (0, D - mW1.shape[1])))
    mb1p = jnp.pad(mb1, (0, D - mb1.shape[0])).reshape(1, D)
    mW2p = jnp.pad(mW2, ((0, D - mW2.shape[0]), (0, D - mW2.shape[1])))
    mb2p = jnp.pad(mb2, (0, D - mb2.shape[0])).reshape(1, D)

    hgat, degpair = _sc_gather_deg(x3, col3, ew3, emb)
    hl1p = _tc_ln_mm(hgat, degpair, g1, c1, W1)
    agg1 = _sc_edge_agg(hl1p, row3, col3, ew3)
    hl2p = _tc_combine_ln_mm(agg1, hl1p, degpair, b1r, g2, c2, W2)
    agg2 = _sc_edge_agg(hl2p, row3, col3, ew3)
    outp = _tc_pool_mlp(agg2, hl2p, degpair, b2r, batchf,
                        mW1p, mb1p, mW2p, mb2p)
    return outp[:, :mW2.shape[1]]


# speedup vs baseline: 18.8941x; 1.0978x over previous
import functools

import jax
import jax.numpy as jnp
from jax import lax
from jax.experimental import pallas as pl
from jax.experimental.pallas import tpu as pltpu
from jax.experimental.pallas import tpu_sc as plsc

N = 10000
E = 320000
D = 128
G = 64

NC = 2
NS = 16
NT = NC * NS

NPAD = 10240
EPAD = 327680
CH = 128
NCH = EPAD // NT // CH
NPT = NPAD // NT
NPS = NPAD // NS
EPT = NCH * CH
BN = 1024


def _mesh():
    return plsc.VectorSubcoreMesh(core_axis_name="c", subcore_axis_name="s")


def _sc_gather_deg(x3, col3, ew3, emb):
    @functools.partial(
        pl.kernel,
        mesh=_mesh(),
        out_type=(
            jax.ShapeDtypeStruct((NPAD, D), jnp.float32),
            jax.ShapeDtypeStruct((NC, NPAD * 16), jnp.float32),
        ),
        scratch_types=[
            pltpu.VMEM((5, 64), jnp.int32),
            pltpu.VMEM((64, D), jnp.float32),
            pltpu.VMEM((64, D), jnp.float32),
            pltpu.VMEM((SUP, CH), jnp.int32),
            pltpu.VMEM((SUP * CH,), jnp.float32),
            pltpu.VMEM((NPS * 16,), jnp.float32),
            pltpu.VMEM_SHARED((NPAD * 16,), jnp.float32),
            pltpu.SemaphoreType.DMA,
            pltpu.SemaphoreType.DMA,
        ],
        compiler_params=pltpu.CompilerParams(needs_layout_passes=False),
    )
    def k(x_hbm, col_hbm, ew_hbm, emb_hbm, h_out, deg_out,
          xall, ga, gb, colall, ewall, zbuf, deg_sh, sem_a, sem_b):
        c = lax.axis_index("c")
        s = lax.axis_index("s")
        tid = c * NS + s

        def zz(i, _):
            zbuf[pl.ds(i * 16, 16)] = jnp.zeros((16,), jnp.float32)
            return 0
        lax.fori_loop(0, NPS, zz, 0)
        pltpu.sync_copy(zbuf, deg_sh.at[pl.ds(s * NPS * 16, NPS * 16)])
        plsc.subcore_barrier()

        pltpu.sync_copy(x_hbm.at[tid], xall)
        bufs = (ga, gb)
        sems = (sem_a, sem_b)
        d = pltpu.async_copy(emb_hbm.at[xall.at[0]], ga, sem_a)
        for t in range(5):
            d.wait()
            if t < 4:
                d = pltpu.async_copy(
                    emb_hbm.at[xall.at[t + 1]], bufs[(t + 1) % 2],
                    sems[(t + 1) % 2])
            pltpu.sync_copy(bufs[t % 2],
                            h_out.at[pl.ds(tid * NPT + t * 64, 64)])

        def super_chunk(ss, _):
            pltpu.sync_copy(col_hbm.at[tid, ss], colall)
            pltpu.sync_copy(ew_hbm.at[tid, ss], ewall)

            def chunk(t, _):
                st = pl.multiple_of(t * CH, CH)
                pltpu.sync_copy(ewall.at[pl.ds(st, CH)],
                                deg_sh.at[colall.at[t]], add=True)
                return 0
            lax.fori_loop(0, SUP, chunk, 0)
            return 0
        lax.fori_loop(0, NSUP, super_chunk, 0)

        plsc.subcore_barrier()
        pltpu.sync_copy(deg_sh.at[pl.ds(s * NPS * 16, NPS * 16)],
                        deg_out.at[c, pl.ds(s * NPS * 16, NPS * 16)])

    return k(x3, col3, ew3, emb)


SUP = 20
NSUP = NCH // SUP


def _sc_edge_agg(hlp, row3, col3, ew3):
    @functools.partial(
        pl.kernel,
        mesh=_mesh(),
        out_type=jax.ShapeDtypeStruct((NC, NPAD, D), jnp.float32),
        scratch_types=[
            pltpu.VMEM((SUP, CH), jnp.int32),
            pltpu.VMEM((SUP, CH), jnp.int32),
            pltpu.VMEM((SUP * CH,), jnp.float32),
            pltpu.VMEM((CH, D), jnp.float32),
            pltpu.VMEM((CH, D), jnp.float32),
            pltpu.VMEM_SHARED((NPAD, D), jnp.float32),
            pltpu.SemaphoreType.DMA,
            pltpu.SemaphoreType.DMA,
        ],
        compiler_params=pltpu.CompilerParams(needs_layout_passes=False),
    )
    def k(hlp_hbm, row_hbm, col_hbm, ew_hbm, out_hbm,
          rowsup, colsup, ewsup, rows_a, rows_b, agg_sh, sem_a, sem_b):
        c = lax.axis_index("c")
        s = lax.axis_index("s")
        tid = c * NS + s

        def zrow(i, _):
            for kk in range(D // 16):
                rows_a[i, pl.ds(16 * kk, 16)] = jnp.zeros((16,), jnp.float32)
            return 0
        lax.fori_loop(0, CH, zrow, 0)
        for i in range(NPS // CH):
            pltpu.sync_copy(rows_a, agg_sh.at[pl.ds(s * NPS + i * CH, CH)])
        plsc.subcore_barrier()

        def scale(rows, t):
            @plsc.parallel_loop(0, CH, unroll=4)
            def _(j):
                fi = jnp.zeros((16,), jnp.int32) + (t * CH + j)
                sv = plsc.load_gather(ewsup, [fi])
                for kk in range(D // 16):
                    rows[j, pl.ds(16 * kk, 16)] = (
                        rows[j, pl.ds(16 * kk, 16)] * sv)

        def super_chunk(ss, _):
            pltpu.sync_copy(row_hbm.at[tid, ss], rowsup)
            pltpu.sync_copy(col_hbm.at[tid, ss], colsup)
            pltpu.sync_copy(ew_hbm.at[tid, ss], ewsup)

            pltpu.async_copy(hlp_hbm.at[rowsup.at[0]], rows_a, sem_a).wait()

            def pair(tt, _):
                t0 = 2 * tt
                t1 = t0 + 1
                t2 = jnp.minimum(t0 + 2, SUP - 1)
                d1 = pltpu.async_copy(
                    hlp_hbm.at[rowsup.at[t1]], rows_b, sem_b)
                scale(rows_a, t0)
                d1.wait()
                pltpu.sync_copy(rows_a, agg_sh.at[colsup.at[t0]], add=True)
                d2 = pltpu.async_copy(
                    hlp_hbm.at[rowsup.at[t2]], rows_a, sem_a)
                scale(rows_b, t1)
                d2.wait()
                pltpu.sync_copy(rows_b, agg_sh.at[colsup.at[t1]], add=True)
                return 0
            lax.fori_loop(0, SUP // 2, pair, 0)
            return 0
        lax.fori_loop(0, NSUP, super_chunk, 0)

        plsc.subcore_barrier()
        pltpu.sync_copy(agg_sh.at[pl.ds(s * NPS, NPS)],
                        out_hbm.at[c, pl.ds(s * NPS, NPS)])

    return k(hlp, row3, col3, ew3)


def _dis_block(degpair):
    deg = 1.0 + degpair[0, :, 0:1] + degpair[1, :, 0:1]
    return lax.rsqrt(deg)


def _ln_block(h, g, b):
    mu = jnp.mean(h, axis=-1, keepdims=True)
    var = jnp.mean((h - mu) ** 2, axis=-1, keepdims=True)
    return (h - mu) / jnp.sqrt(var + 1e-5) * g + b


def _mm(a, b):
    return lax.dot_general(a, b, (((1,), (0,)), ((), ())),
                           precision=lax.Precision.HIGHEST,
                           preferred_element_type=jnp.float32)


def _tc_ln_mm(hgat, degpair, g1, b1, W1):
    def body(h_ref, deg_ref, g_ref, b_ref, w_ref, o_ref):
        dis = _dis_block(deg_ref[...])
        hn = _ln_block(h_ref[...], g_ref[...], b_ref[...])
        o_ref[...] = dis * _mm(hn, w_ref[...])

    return pl.pallas_call(
        body,
        grid=(NPAD // BN,),
        in_specs=[
            pl.BlockSpec((BN, D), lambda i: (i, 0)),
            pl.BlockSpec((NC, BN, 16), lambda i: (0, i, 0)),
            pl.BlockSpec((1, D), lambda i: (0, 0)),
            pl.BlockSpec((1, D), lambda i: (0, 0)),
            pl.BlockSpec((D, D), lambda i: (0, 0)),
        ],
        out_specs=pl.BlockSpec((BN, D), lambda i: (i, 0)),
        out_shape=jax.ShapeDtypeStruct((NPAD, D), jnp.float32),
    )(hgat, degpair, g1, b1, W1)


def _tc_combine_ln_mm(aggpair, hlp, degpair, bias, g2, b2, W2):
    def body(a_ref, hlp_ref, deg_ref, bias_ref, g_ref, b_ref, w_ref, o_ref):
        dis = _dis_block(deg_ref[...])
        a = a_ref[0] + a_ref[1] + hlp_ref[...]
        h2 = jnp.maximum(dis * a + bias_ref[...], 0.0)
        hn = _ln_block(h2, g_ref[...], b_ref[...])
        o_ref[...] = dis * _mm(hn, w_ref[...])

    return pl.pallas_call(
        body,
        grid=(NPAD // BN,),
        in_specs=[
            pl.BlockSpec((NC, BN, D), lambda i: (0, i, 0)),
            pl.BlockSpec((BN, D), lambda i: (i, 0)),
            pl.BlockSpec((NC, BN, 16), lambda i: (0, i, 0)),
            pl.BlockSpec((1, D), lambda i: (0, 0)),
            pl.BlockSpec((1, D), lambda i: (0, 0)),
            pl.BlockSpec((1, D), lambda i: (0, 0)),
            pl.BlockSpec((D, D), lambda i: (0, 0)),
        ],
        out_specs=pl.BlockSpec((BN, D), lambda i: (i, 0)),
        out_shape=jax.ShapeDtypeStruct((NPAD, D), jnp.float32),
    )(aggpair, hlp, degpair, bias, g2, b2, W2)


def _tc_pool_mlp(aggpair, hlp, degpair, bias, batchf, mW1p, mb1p, mW2p, mb2p):
    nb = NPAD // BN

    def body(a_ref, hlp_ref, deg_ref, bias_ref, bt_ref,
             w1_ref, c1_ref, w2_ref, c2_ref, o_ref, pool_ref, cnt_ref):
        i = pl.program_id(0)

        @pl.when(i == 0)
        def _():
            pool_ref[...] = jnp.zeros_like(pool_ref)
            cnt_ref[...] = jnp.zeros_like(cnt_ref)

        dis = _dis_block(deg_ref[...])
        a = a_ref[0] + a_ref[1] + hlp_ref[...]
        h3 = jnp.maximum(dis * a + bias_ref[...], 0.0)

        bt = bt_ref[0]
        gid = lax.broadcasted_iota(jnp.int32, (G, BN), 0)
        onehot = jnp.where(gid == bt, 1.0, 0.0)
        pool_ref[...] += _mm(onehot, h3)
        cnt_ref[...] += jnp.sum(onehot, axis=1, keepdims=True)

        @pl.when(i == nb - 1)
        def _():
            gavg = pool_ref[...] / jnp.maximum(cnt_ref[...], 1.0)
            z = jnp.maximum(_mm(gavg, w1_ref[...]) + c1_ref[...], 0.0)
            o_ref[...] = _mm(z, w2_ref[...]) + c2_ref[...]

    return pl.pallas_call(
        body,
        grid=(nb,),
        in_specs=[
            pl.BlockSpec((NC, BN, D), lambda i: (0, i, 0)),
            pl.BlockSpec((BN, D), lambda i: (i, 0)),
            pl.BlockSpec((NC, BN, 16), lambda i: (0, i, 0)),
            pl.BlockSpec((1, D), lambda i: (0, 0)),
            pl.BlockSpec((1, 1, BN), lambda i: (i, 0, 0)),
            pl.BlockSpec((D, D), lambda i: (0, 0)),
            pl.BlockSpec((1, D), lambda i: (0, 0)),
            pl.BlockSpec((D, D), lambda i: (0, 0)),
            pl.BlockSpec((1, D), lambda i: (0, 0)),
        ],
        out_specs=pl.BlockSpec((G, D), lambda i: (0, 0)),
        out_shape=jax.ShapeDtypeStruct((G, D), jnp.float32),
        scratch_shapes=[
            pltpu.VMEM((G, D), jnp.float32),
            pltpu.VMEM((G, 1), jnp.float32),
        ],
    )(aggpair, hlp, degpair, bias, batchf, mW1p, mb1p, mW2p, mb2p)


def kernel(x, edge_index, edge_attr, batch, emb, ln1_g, ln1_b, W1, b1,
           ln2_g, ln2_b, W2, b2, mW1, mb1, mW2, mb2):
    i32 = jnp.int32
    f32 = jnp.float32

    xspread = jnp.arange(NPAD - N, dtype=i32) % emb.shape[0]
    x3 = jnp.concatenate([x.astype(i32), xspread]).reshape(NT, 5, 64)
    row = edge_index[0].astype(i32)
    col = edge_index[1].astype(i32)
    epad = EPAD - E
    spread = jnp.arange(epad, dtype=i32) % N
    colp = jnp.concatenate([col, spread])
    row3 = jnp.concatenate([row, spread]).reshape(NT, NSUP, SUP, CH)
    col3 = colp.reshape(NT, NSUP, SUP, CH)
    col16 = (colp * 16).reshape(NT, NSUP, SUP, CH)
    ew3 = jnp.concatenate(
        [edge_attr.astype(f32), jnp.zeros((epad,), f32)]
    ).reshape(NT, NSUP, SUP * CH)
    batchf = jnp.concatenate(
        [batch.astype(i32), jnp.full((NPAD - N,), 65536, i32)]
    ).reshape(NPAD // BN, 1, BN)

    g1 = ln1_g.reshape(1, D)
    c1 = ln1_b.reshape(1, D)
    g2 = ln2_g.reshape(1, D)
    c2 = ln2_b.reshape(1, D)
    b1r = b1.reshape(1, D)
    b2r = b2.reshape(1, D)
    mW1p = jnp.pad(mW1, ((0, 0), (0, D - mW1.shape[1])))
    mb1p = jnp.pad(mb1, (0, D - mb1.shape[0])).reshape(1, D)
    mW2p = jnp.pad(mW2, ((0, D - mW2.shape[0]), (0, D - mW2.shape[1])))
    mb2p = jnp.pad(mb2, (0, D - mb2.shape[0])).reshape(1, D)

    hgat, degflat = _sc_gather_deg(x3, col16, ew3, emb)
    degpair = degflat.reshape(NC, NPAD, 16)
    hl1p = _tc_ln_mm(hgat, degpair, g1, c1, W1)
    agg1 = _sc_edge_agg(hl1p, row3, col3, ew3)
    hl2p = _tc_combine_ln_mm(agg1, hl1p, degpair, b1r, g2, c2, W2)
    agg2 = _sc_edge_agg(hl2p, row3, col3, ew3)
    outp = _tc_pool_mlp(agg2, hl2p, degpair, b2r, batchf,
                        mW1p, mb1p, mW2p, mb2p)
    return outp[:, :mW2.shape[1]]


# speedup vs baseline: 20.4698x; 1.0834x over previous
import functools

import jax
import jax.numpy as jnp
from jax import lax
from jax.experimental import pallas as pl
from jax.experimental.pallas import tpu as pltpu
from jax.experimental.pallas import tpu_sc as plsc

N = 10000
E = 320000
D = 128
G = 64

NC = 2
NS = 16
NT = NC * NS

NPAD = 10240
EPAD = 327680
CH = 128
NCH = EPAD // NT // CH
NPT = NPAD // NT
NPS = NPAD // NS
EPT = NCH * CH
BN = 1024


def _mesh():
    return plsc.VectorSubcoreMesh(core_axis_name="c", subcore_axis_name="s")


def _sc_gather_deg(x3, col3, ew3, emb):
    @functools.partial(
        pl.kernel,
        mesh=_mesh(),
        out_type=(
            jax.ShapeDtypeStruct((NPAD, D), jnp.float32),
            jax.ShapeDtypeStruct((NC, NPAD * 16), jnp.float32),
        ),
        scratch_types=[
            pltpu.VMEM((5, 64), jnp.int32),
            pltpu.VMEM((64, D), jnp.float32),
            pltpu.VMEM((64, D), jnp.float32),
            pltpu.VMEM((SUP, CH), jnp.int32),
            pltpu.VMEM((SUP * CH,), jnp.float32),
            pltpu.VMEM((NPS * 16,), jnp.float32),
            pltpu.VMEM_SHARED((NPAD * 16,), jnp.float32),
            pltpu.SemaphoreType.DMA,
            pltpu.SemaphoreType.DMA,
        ],
        compiler_params=pltpu.CompilerParams(needs_layout_passes=False),
    )
    def k(x_hbm, col_hbm, ew_hbm, emb_hbm, h_out, deg_out,
          xall, ga, gb, colall, ewall, zbuf, deg_sh, sem_a, sem_b):
        c = lax.axis_index("c")
        s = lax.axis_index("s")
        tid = c * NS + s

        def zz(i, _):
            zbuf[pl.ds(i * 16, 16)] = jnp.zeros((16,), jnp.float32)
            return 0
        lax.fori_loop(0, NPS, zz, 0)
        pltpu.sync_copy(zbuf, deg_sh.at[pl.ds(s * NPS * 16, NPS * 16)])
        plsc.subcore_barrier()

        pltpu.sync_copy(x_hbm.at[tid], xall)
        bufs = (ga, gb)
        sems = (sem_a, sem_b)
        d = pltpu.async_copy(emb_hbm.at[xall.at[0]], ga, sem_a)
        for t in range(5):
            d.wait()
            if t < 4:
                d = pltpu.async_copy(
                    emb_hbm.at[xall.at[t + 1]], bufs[(t + 1) % 2],
                    sems[(t + 1) % 2])
            pltpu.sync_copy(bufs[t % 2],
                            h_out.at[pl.ds(tid * NPT + t * 64, 64)])

        def super_chunk(ss, _):
            pltpu.sync_copy(col_hbm.at[tid, ss], colall)
            pltpu.sync_copy(ew_hbm.at[tid, ss], ewall)

            def chunk(t, _):
                st = pl.multiple_of(t * CH, CH)
                pltpu.sync_copy(ewall.at[pl.ds(st, CH)],
                                deg_sh.at[colall.at[t]], add=True)
                return 0
            lax.fori_loop(0, SUP, chunk, 0)
            return 0
        lax.fori_loop(0, NSUP, super_chunk, 0)

        plsc.subcore_barrier()
        pltpu.sync_copy(deg_sh.at[pl.ds(s * NPS * 16, NPS * 16)],
                        deg_out.at[c, pl.ds(s * NPS * 16, NPS * 16)])

    return k(x3, col3, ew3, emb)


SUP = 20
NSUP = NCH // SUP


def _sc_edge_agg(hlp, row3, col3, ew3):
    @functools.partial(
        pl.kernel,
        mesh=_mesh(),
        out_type=jax.ShapeDtypeStruct((NC, NPAD, D), jnp.float32),
        scratch_types=[
            pltpu.VMEM((SUP, CH), jnp.int32),
            pltpu.VMEM((SUP, CH), jnp.int32),
            pltpu.VMEM((SUP * CH,), jnp.float32),
            pltpu.VMEM((CH, D), jnp.float32),
            pltpu.VMEM((CH, D), jnp.float32),
            pltpu.VMEM_SHARED((NPAD, D), jnp.float32),
            pltpu.SemaphoreType.DMA,
            pltpu.SemaphoreType.DMA,
            pltpu.SemaphoreType.DMA,
            pltpu.SemaphoreType.DMA,
        ],
        compiler_params=pltpu.CompilerParams(needs_layout_passes=False),
    )
    def k(hlp_hbm, row_hbm, col_hbm, ew_hbm, out_hbm,
          rowsup, colsup, ewsup, rows_a, rows_b, agg_sh,
          sem_a, sem_b, sem_s0, sem_s1):
        c = lax.axis_index("c")
        s = lax.axis_index("s")
        tid = c * NS + s

        def zrow(i, _):
            for kk in range(D // 16):
                rows_a[i, pl.ds(16 * kk, 16)] = jnp.zeros((16,), jnp.float32)
            return 0
        lax.fori_loop(0, CH, zrow, 0)
        for i in range(NPS // CH):
            pltpu.sync_copy(rows_a, agg_sh.at[pl.ds(s * NPS + i * CH, CH)])
        plsc.subcore_barrier()

        def scale(rows, t):
            @plsc.parallel_loop(0, CH, unroll=4)
            def _(j):
                fi = jnp.zeros((16,), jnp.int32) + (t * CH + j)
                sv = plsc.load_gather(ewsup, [fi])
                for kk in range(D // 16):
                    rows[j, pl.ds(16 * kk, 16)] = (
                        rows[j, pl.ds(16 * kk, 16)] * sv)

        def super_chunk(ss, _):
            pltpu.sync_copy(row_hbm.at[tid, ss], rowsup)
            pltpu.sync_copy(col_hbm.at[tid, ss], colsup)
            pltpu.sync_copy(ew_hbm.at[tid, ss], ewsup)

            pltpu.async_copy(hlp_hbm.at[rowsup.at[0]], rows_a, sem_a).wait()

            def pair(tt, _):
                t0 = 2 * tt
                t1 = t0 + 1
                t2 = jnp.minimum(t0 + 2, SUP - 1)
                d1 = pltpu.async_copy(
                    hlp_hbm.at[rowsup.at[t1]], rows_b, sem_b)
                scale(rows_a, t0)
                d1.wait()
                s0 = pltpu.async_copy(
                    rows_a, agg_sh.at[colsup.at[t0]], sem_s0, add=True)
                scale(rows_b, t1)
                s0.wait()
                d2 = pltpu.async_copy(
                    hlp_hbm.at[rowsup.at[t2]], rows_a, sem_a)
                s1 = pltpu.async_copy(
                    rows_b, agg_sh.at[colsup.at[t1]], sem_s1, add=True)
                d2.wait()
                s1.wait()
                return 0
            lax.fori_loop(0, SUP // 2, pair, 0)
            return 0
        lax.fori_loop(0, NSUP, super_chunk, 0)

        plsc.subcore_barrier()
        pltpu.sync_copy(agg_sh.at[pl.ds(s * NPS, NPS)],
                        out_hbm.at[c, pl.ds(s * NPS, NPS)])

    return k(hlp, row3, col3, ew3)


def _dis_block(degpair):
    deg = 1.0 + degpair[0, :, 0:1] + degpair[1, :, 0:1]
    return lax.rsqrt(deg)


def _ln_block(h, g, b):
    mu = jnp.mean(h, axis=-1, keepdims=True)
    var = jnp.mean((h - mu) ** 2, axis=-1, keepdims=True)
    return (h - mu) / jnp.sqrt(var + 1e-5) * g + b


def _mm(a, b):
    return lax.dot_general(a, b, (((1,), (0,)), ((), ())),
                           precision=lax.Precision.HIGHEST,
                           preferred_element_type=jnp.float32)


def _tc_ln_mm(hgat, degpair, g1, b1, W1):
    def body(h_ref, deg_ref, g_ref, b_ref, w_ref, o_ref):
        dis = _dis_block(deg_ref[...])
        hn = _ln_block(h_ref[...], g_ref[...], b_ref[...])
        o_ref[...] = dis * _mm(hn, w_ref[...])

    return pl.pallas_call(
        body,
        grid=(NPAD // BN,),
        in_specs=[
            pl.BlockSpec((BN, D), lambda i: (i, 0)),
            pl.BlockSpec((NC, BN, 16), lambda i: (0, i, 0)),
            pl.BlockSpec((1, D), lambda i: (0, 0)),
            pl.BlockSpec((1, D), lambda i: (0, 0)),
            pl.BlockSpec((D, D), lambda i: (0, 0)),
        ],
        out_specs=pl.BlockSpec((BN, D), lambda i: (i, 0)),
        out_shape=jax.ShapeDtypeStruct((NPAD, D), jnp.float32),
    )(hgat, degpair, g1, b1, W1)


def _tc_combine_ln_mm(aggpair, hlp, degpair, bias, g2, b2, W2):
    def body(a_ref, hlp_ref, deg_ref, bias_ref, g_ref, b_ref, w_ref, o_ref):
        dis = _dis_block(deg_ref[...])
        a = a_ref[0] + a_ref[1] + hlp_ref[...]
        h2 = jnp.maximum(dis * a + bias_ref[...], 0.0)
        hn = _ln_block(h2, g_ref[...], b_ref[...])
        o_ref[...] = dis * _mm(hn, w_ref[...])

    return pl.pallas_call(
        body,
        grid=(NPAD // BN,),
        in_specs=[
            pl.BlockSpec((NC, BN, D), lambda i: (0, i, 0)),
            pl.BlockSpec((BN, D), lambda i: (i, 0)),
            pl.BlockSpec((NC, BN, 16), lambda i: (0, i, 0)),
            pl.BlockSpec((1, D), lambda i: (0, 0)),
            pl.BlockSpec((1, D), lambda i: (0, 0)),
            pl.BlockSpec((1, D), lambda i: (0, 0)),
            pl.BlockSpec((D, D), lambda i: (0, 0)),
        ],
        out_specs=pl.BlockSpec((BN, D), lambda i: (i, 0)),
        out_shape=jax.ShapeDtypeStruct((NPAD, D), jnp.float32),
    )(aggpair, hlp, degpair, bias, g2, b2, W2)


def _tc_pool_mlp(aggpair, hlp, degpair, bias, batchf, mW1p, mb1p, mW2p, mb2p):
    nb = NPAD // BN

    def body(a_ref, hlp_ref, deg_ref, bias_ref, bt_ref,
             w1_ref, c1_ref, w2_ref, c2_ref, o_ref, pool_ref, cnt_ref):
        i = pl.program_id(0)

        @pl.when(i == 0)
        def _():
            pool_ref[...] = jnp.zeros_like(pool_ref)
            cnt_ref[...] = jnp.zeros_like(cnt_ref)

        dis = _dis_block(deg_ref[...])
        a = a_ref[0] + a_ref[1] + hlp_ref[...]
        h3 = jnp.maximum(dis * a + bias_ref[...], 0.0)

        bt = bt_ref[0]
        gid = lax.broadcasted_iota(jnp.int32, (G, BN), 0)
        onehot = jnp.where(gid == bt, 1.0, 0.0)
        pool_ref[...] += _mm(onehot, h3)
        cnt_ref[...] += jnp.sum(onehot, axis=1, keepdims=True)

        @pl.when(i == nb - 1)
        def _():
            gavg = pool_ref[...] / jnp.maximum(cnt_ref[...], 1.0)
            z = jnp.maximum(_mm(gavg, w1_ref[...]) + c1_ref[...], 0.0)
            o_ref[...] = _mm(z, w2_ref[...]) + c2_ref[...]

    return pl.pallas_call(
        body,
        grid=(nb,),
        in_specs=[
            pl.BlockSpec((NC, BN, D), lambda i: (0, i, 0)),
            pl.BlockSpec((BN, D), lambda i: (i, 0)),
            pl.BlockSpec((NC, BN, 16), lambda i: (0, i, 0)),
            pl.BlockSpec((1, D), lambda i: (0, 0)),
            pl.BlockSpec((1, 1, BN), lambda i: (i, 0, 0)),
            pl.BlockSpec((D, D), lambda i: (0, 0)),
            pl.BlockSpec((1, D), lambda i: (0, 0)),
            pl.BlockSpec((D, D), lambda i: (0, 0)),
            pl.BlockSpec((1, D), lambda i: (0, 0)),
        ],
        out_specs=pl.BlockSpec((G, D), lambda i: (0, 0)),
        out_shape=jax.ShapeDtypeStruct((G, D), jnp.float32),
        scratch_shapes=[
            pltpu.VMEM((G, D), jnp.float32),
            pltpu.VMEM((G, 1), jnp.float32),
        ],
    )(aggpair, hlp, degpair, bias, batchf, mW1p, mb1p, mW2p, mb2p)


def kernel(x, edge_index, edge_attr, batch, emb, ln1_g, ln1_b, W1, b1,
           ln2_g, ln2_b, W2, b2, mW1, mb1, mW2, mb2):
    i32 = jnp.int32
    f32 = jnp.float32

    xspread = jnp.arange(NPAD - N, dtype=i32) % emb.shape[0]
    x3 = jnp.concatenate([x.astype(i32), xspread]).reshape(NT, 5, 64)
    row = edge_index[0].astype(i32)
    col = edge_index[1].astype(i32)
    epad = EPAD - E
    spread = jnp.arange(epad, dtype=i32) % N
    colp = jnp.concatenate([col, spread])
    row3 = jnp.concatenate([row, spread]).reshape(NT, NSUP, SUP, CH)
    col3 = colp.reshape(NT, NSUP, SUP, CH)
    col16 = (colp * 16).reshape(NT, NSUP, SUP, CH)
    ew3 = jnp.concatenate(
        [edge_attr.astype(f32), jnp.zeros((epad,), f32)]
    ).reshape(NT, NSUP, SUP * CH)
    batchf = jnp.concatenate(
        [batch.astype(i32), jnp.full((NPAD - N,), 65536, i32)]
    ).reshape(NPAD // BN, 1, BN)

    g1 = ln1_g.reshape(1, D)
    c1 = ln1_b.reshape(1, D)
    g2 = ln2_g.reshape(1, D)
    c2 = ln2_b.reshape(1, D)
    b1r = b1.reshape(1, D)
    b2r = b2.reshape(1, D)
    mW1p = jnp.pad(mW1, ((0, 0), (0, D - mW1.shape[1])))
    mb1p = jnp.pad(mb1, (0, D - mb1.shape[0])).reshape(1, D)
    mW2p = jnp.pad(mW2, ((0, D - mW2.shape[0]), (0, D - mW2.shape[1])))
    mb2p = jnp.pad(mb2, (0, D - mb2.shape[0])).reshape(1, D)

    hgat, degflat = _sc_gather_deg(x3, col16, ew3, emb)
    degpair = degflat.reshape(NC, NPAD, 16)
    hl1p = _tc_ln_mm(hgat, degpair, g1, c1, W1)
    agg1 = _sc_edge_agg(hl1p, row3, col3, ew3)
    hl2p = _tc_combine_ln_mm(agg1, hl1p, degpair, b1r, g2, c2, W2)
    agg2 = _sc_edge_agg(hl2p, row3, col3, ew3)
    outp = _tc_pool_mlp(agg2, hl2p, degpair, b2r, batchf,
                        mW1p, mb1p, mW2p, mb2p)
    return outp[:, :mW2.shape[1]]
